# Initial kernel scaffold; baseline (speedup 1.0000x reference)
import jax
import jax.numpy as jnp
from jax.experimental import pallas as pl


def kernel(z_groups, embedding_weight):
    raise NotImplementedError("write your pallas kernel here")


# speedup vs baseline: 58.8283x; 58.8283x over previous
import jax
import jax.numpy as jnp
from jax.experimental import pallas as pl
from jax.experimental.pallas import tpu as pltpu

N_E = 8192
E_DIM = 64
GROUPS = 4
NG = N_E // GROUPS
ROWS = 16 * 32 * 32
T = 256
NT = ROWS // T


def _vq_body(z_ref, emb_ref, quant_ref, idx_ref, me_ref, perp_ref, counts_ref):
    i = pl.program_id(0)

    @pl.when(i == 0)
    def _init():
        counts_ref[...] = jnp.zeros_like(counts_ref)

    emb = emb_ref[...]
    for g in range(GROUPS):
        z = z_ref[:, g * E_DIM:(g + 1) * E_DIM]
        zn = z / jnp.maximum(
            jnp.sqrt(jnp.sum(z * z, axis=1, keepdims=True)), 1e-12)
        e = emb[g * NG:(g + 1) * NG, :]
        en = e / jnp.maximum(
            jnp.sqrt(jnp.sum(e * e, axis=1, keepdims=True)), 1e-12)

        s = jax.lax.dot_general(zn, en, (((1,), (1,)), ((), ())),
                                preferred_element_type=jnp.float32)
        zsq = jnp.sum(zn * zn, axis=1, keepdims=True)
        esq = jnp.sum(en * en, axis=1, keepdims=True)
        d = -zsq - esq.reshape(1, NG) + 2.0 * s

        m = jnp.max(d, axis=1, keepdims=True)
        lane = jax.lax.broadcasted_iota(jnp.int32, (T, NG), 1)
        idx = jnp.max(jnp.where(d >= m, lane, -1), axis=1, keepdims=True)

        oh = (lane == idx).astype(jnp.float32)
        zq = jax.lax.dot_general(oh, en, (((1,), (0,)), ((), ())),
                                 preferred_element_type=jnp.float32)
        quant_ref[:, g * E_DIM:(g + 1) * E_DIM] = zq
        idx_ref[:, g:g + 1] = idx
        if g == GROUPS - 1:
            me_ref[...] = oh
            counts_ref[...] += jnp.sum(oh, axis=0, keepdims=True)

    @pl.when(i == NT - 1)
    def _finish():
        avg = counts_ref[...] / float(ROWS)
        ent = jnp.sum(avg * jnp.log(avg + 1e-10), axis=1, keepdims=True)
        perp_ref[...] = jnp.exp(-ent)


_vq_call = pl.pallas_call(
    _vq_body,
    grid=(NT,),
    in_specs=[
        pl.BlockSpec((T, GROUPS * E_DIM), lambda i: (i, 0)),
        pl.BlockSpec((N_E, E_DIM), lambda i: (0, 0)),
    ],
    out_specs=[
        pl.BlockSpec((T, GROUPS * E_DIM), lambda i: (i, 0)),
        pl.BlockSpec((T, GROUPS), lambda i: (i, 0)),
        pl.BlockSpec((T, NG), lambda i: (i, 0)),
        pl.BlockSpec((1, 1), lambda i: (0, 0)),
    ],
    out_shape=[
        jax.ShapeDtypeStruct((ROWS, GROUPS * E_DIM), jnp.float32),
        jax.ShapeDtypeStruct((ROWS, GROUPS), jnp.int32),
        jax.ShapeDtypeStruct((ROWS, NG), jnp.float32),
        jax.ShapeDtypeStruct((1, 1), jnp.float32),
    ],
    scratch_shapes=[pltpu.VMEM((1, NG), jnp.float32)],
)


def kernel(z_groups, embedding_weight):
    b = z_groups.shape[0]
    z2d = z_groups.transpose(0, 2, 3, 1).reshape(ROWS, GROUPS * E_DIM)
    quant2d, idx, me, perp = _vq_call(z2d, embedding_weight)
    quant = quant2d.reshape(b, 32, 32, GROUPS * E_DIM).transpose(0, 3, 1, 2)
    zeros_g = jnp.zeros((GROUPS,), jnp.float32)
    return (quant, zeros_g, zeros_g, zeros_g, perp[0, 0], me, idx)


# speedup vs baseline: 90.5513x; 1.5392x over previous
import jax
import jax.numpy as jnp
from jax.experimental import pallas as pl
from jax.experimental.pallas import tpu as pltpu

N_E = 8192
E_DIM = 64
GROUPS = 4
NG = N_E // GROUPS
ROWS = 16 * 32 * 32
T = 512
NT = ROWS // T


def _vq_body(z_ref, emb_ref, quant_ref, idx_ref, me_ref, perp_ref,
             counts_ref, en_ref, esq_ref):
    i = pl.program_id(0)

    @pl.when(i == 0)
    def _init():
        counts_ref[...] = jnp.zeros_like(counts_ref)
        emb = emb_ref[...]
        for g in range(GROUPS):
            e = emb[g * NG:(g + 1) * NG, :]
            en = e / jnp.maximum(
                jnp.sqrt(jnp.sum(e * e, axis=1, keepdims=True)), 1e-12)
            en_ref[g * NG:(g + 1) * NG, :] = en
            esq = jnp.sum(en * en, axis=1, keepdims=True)
            esq_ref[g:g + 1, :] = esq.reshape(1, NG)

    for g in range(GROUPS):
        z = z_ref[:, g * E_DIM:(g + 1) * E_DIM]
        zn = z / jnp.maximum(
            jnp.sqrt(jnp.sum(z * z, axis=1, keepdims=True)), 1e-12)
        en = en_ref[g * NG:(g + 1) * NG, :]

        s = jax.lax.dot_general(zn, en, (((1,), (1,)), ((), ())),
                                preferred_element_type=jnp.float32)
        zsq = jnp.sum(zn * zn, axis=1, keepdims=True)
        d = -zsq - esq_ref[g:g + 1, :] + 2.0 * s

        m = jnp.max(d, axis=1, keepdims=True)
        lane = jax.lax.broadcasted_iota(jnp.int32, (T, NG), 1)
        idx = jnp.max(jnp.where(d >= m, lane, -1), axis=1, keepdims=True)

        oh = (lane == idx).astype(jnp.float32)
        zq = jax.lax.dot_general(oh, en, (((1,), (0,)), ((), ())),
                                 preferred_element_type=jnp.float32)
        quant_ref[:, g * E_DIM:(g + 1) * E_DIM] = zq
        idx_ref[:, g:g + 1] = idx
        if g == GROUPS - 1:
            me_ref[...] = oh
            counts_ref[...] += jnp.sum(oh, axis=0, keepdims=True)

    @pl.when(i == NT - 1)
    def _finish():
        avg = counts_ref[...] / float(ROWS)
        ent = jnp.sum(avg * jnp.log(avg + 1e-10), axis=1, keepdims=True)
        perp_ref[...] = jnp.exp(-ent)


_vq_call = pl.pallas_call(
    _vq_body,
    grid=(NT,),
    in_specs=[
        pl.BlockSpec((T, GROUPS * E_DIM), lambda i: (i, 0)),
        pl.BlockSpec((N_E, E_DIM), lambda i: (0, 0)),
    ],
    out_specs=[
        pl.BlockSpec((T, GROUPS * E_DIM), lambda i: (i, 0)),
        pl.BlockSpec((T, GROUPS), lambda i: (i, 0)),
        pl.BlockSpec((T, NG), lambda i: (i, 0)),
        pl.BlockSpec((1, 1), lambda i: (0, 0)),
    ],
    out_shape=[
        jax.ShapeDtypeStruct((ROWS, GROUPS * E_DIM), jnp.float32),
        jax.ShapeDtypeStruct((ROWS, GROUPS), jnp.int32),
        jax.ShapeDtypeStruct((ROWS, NG), jnp.float32),
        jax.ShapeDtypeStruct((1, 1), jnp.float32),
    ],
    scratch_shapes=[
        pltpu.VMEM((1, NG), jnp.float32),
        pltpu.VMEM((N_E, E_DIM), jnp.float32),
        pltpu.VMEM((GROUPS, NG), jnp.float32),
    ],
)


def kernel(z_groups, embedding_weight):
    b = z_groups.shape[0]
    z2d = z_groups.transpose(0, 2, 3, 1).reshape(ROWS, GROUPS * E_DIM)
    quant2d, idx, me, perp = _vq_call(z2d, embedding_weight)
    quant = quant2d.reshape(b, 32, 32, GROUPS * E_DIM).transpose(0, 3, 1, 2)
    zeros_g = jnp.zeros((GROUPS,), jnp.float32)
    return (quant, zeros_g, zeros_g, zeros_g, perp[0, 0], me, idx)
